# baseline (speedup 1.0000x reference)
import functools

import jax
import jax.numpy as jnp
from jax import lax
from jax.experimental import pallas as pl
from jax.experimental.pallas import tpu as pltpu
from jax.experimental.pallas import tpu_sc as plsc

_NC = 2
_NS = 16
_NW = _NC * _NS

_BATCH = 4
_SEQ = 8192
_B = _BATCH * _SEQ
_D = 1024
_BPW = _B // _NW
_WPB = _SEQ // _BPW
_C = 32
_NCHUNK = _BPW // _C


def _make_sc_gather():
    mesh = plsc.VectorSubcoreMesh(core_axis_name="c", subcore_axis_name="s")

    @functools.partial(
        pl.kernel,
        mesh=mesh,
        out_type=jax.ShapeDtypeStruct((_B, _D), jnp.float32),
        scratch_types=[
            pltpu.VMEM((_BPW,), jnp.int32),
            pltpu.VMEM((_C, _D), jnp.float32),
            pltpu.VMEM((_C, _D), jnp.float32),
            pltpu.SemaphoreType.DMA,
            pltpu.SemaphoreType.DMA,
        ],
    )
    def gather_kernel(table_hbm, idx_hbm, out_hbm, idx_v, buf0, buf1, gsem, wsem):
        wid = lax.axis_index("s") * _NC + lax.axis_index("c")
        base = wid * _BPW
        pltpu.sync_copy(
            idx_hbm.at[wid // _WPB, pl.ds((wid % _WPB) * _BPW, _BPW)], idx_v
        )
        bufs = (buf0, buf1)
        gathers = [None, None]
        writes = [None, None]

        def start_gather(c):
            return pltpu.async_copy(
                table_hbm.at[idx_v.at[pl.ds(c * _C, _C)]], bufs[c % 2], gsem
            )

        gathers[0] = start_gather(0)
        for c in range(_NCHUNK):
            if c >= 1:
                writes[(c - 1) % 2].wait()
            if c + 1 < _NCHUNK:
                gathers[(c + 1) % 2] = start_gather(c + 1)
            gathers[c % 2].wait()
            writes[c % 2] = pltpu.async_copy(
                bufs[c % 2], out_hbm.at[pl.ds(base + c * _C, _C)], wsem
            )
        writes[(_NCHUNK - 1) % 2].wait()

    return gather_kernel


_sc_gather = _make_sc_gather()


def kernel(x, table):
    out = _sc_gather(table, x)
    return out.reshape(x.shape + (table.shape[1],))


# speedup vs baseline: 1.0049x; 1.0049x over previous
import functools

import jax
import jax.numpy as jnp
from jax import lax
from jax.experimental import pallas as pl
from jax.experimental.pallas import tpu as pltpu
from jax.experimental.pallas import tpu_sc as plsc

_NC = 2
_NS = 16
_NW = _NC * _NS

_BATCH = 4
_SEQ = 8192
_B = _BATCH * _SEQ
_D = 1024
_BPW = _B // _NW
_WPB = _SEQ // _BPW
_C = 16
_NCHUNK = _BPW // _C


def _make_sc_gather():
    mesh = plsc.VectorSubcoreMesh(core_axis_name="c", subcore_axis_name="s")

    @functools.partial(
        pl.kernel,
        mesh=mesh,
        out_type=jax.ShapeDtypeStruct((_B, _D), jnp.float32),
        scratch_types=[
            pltpu.VMEM((_BPW,), jnp.int32),
            pltpu.VMEM((_C, _D), jnp.float32),
            pltpu.VMEM((_C, _D), jnp.float32),
            pltpu.VMEM((_C, _D), jnp.float32),
            pltpu.VMEM((_C, _D), jnp.float32),
            pltpu.SemaphoreType.DMA,
            pltpu.SemaphoreType.DMA,
        ],
    )
    def gather_kernel(table_hbm, idx_hbm, out_hbm, idx_v, buf0, buf1, buf2, buf3, gsem, wsem):
        wid = lax.axis_index("s") * _NC + lax.axis_index("c")
        base = wid * _BPW
        pltpu.sync_copy(
            idx_hbm.at[wid // _WPB, pl.ds((wid % _WPB) * _BPW, _BPW)], idx_v
        )
        bufs = (buf0, buf1, buf2, buf3)
        gathers = [None] * 4
        writes = [None] * 4

        def start_gather(c):
            return pltpu.async_copy(
                table_hbm.at[idx_v.at[pl.ds(c * _C, _C)]], bufs[c % 4], gsem
            )

        gathers[0] = start_gather(0)
        gathers[1] = start_gather(1)
        for c in range(_NCHUNK):
            if c >= 2:
                writes[(c - 2) % 4].wait()
            if c + 2 < _NCHUNK:
                gathers[(c + 2) % 4] = start_gather(c + 2)
            gathers[c % 4].wait()
            writes[c % 4] = pltpu.async_copy(
                bufs[c % 4], out_hbm.at[pl.ds(base + c * _C, _C)], wsem
            )
        writes[(_NCHUNK - 2) % 4].wait()
        writes[(_NCHUNK - 1) % 4].wait()

    return gather_kernel


_sc_gather = _make_sc_gather()


def kernel(x, table):
    out = _sc_gather(table, x)
    return out.reshape(x.shape + (table.shape[1],))
